# baseline (speedup 1.0000x reference)
import functools

import jax
import jax.numpy as jnp
from jax import lax
from jax.experimental import pallas as pl
from jax.experimental.pallas import tpu as pltpu
from jax.experimental.pallas import tpu_sc as plsc

_BATCH = 16384
_D = 32
_LAMBDA = 0.01
_NU = 1000000
_NI = 100000

_NC = 2
_NS = 16
_NW = _NC * _NS
_BPW = _BATCH // _NW
_GCHUNK = 128
_NCHUNK = _BPW // _GCHUNK
_PITCH = _BPW + 1


def _mse_sc_body(u_hbm, i_hbm, r_hbm, upk, ipk, out_hbm,
                 u_idx, i_idx, r_v, u_rows, i_rows, tbuf,
                 shared, parts_v, stage_v, sem):
    c = lax.axis_index("c")
    s = lax.axis_index("s")
    wid = c * _NS + s
    base = wid * _BPW

    pltpu.sync_copy(u_hbm.at[pl.ds(base, _BPW)], u_idx)
    pltpu.sync_copy(i_hbm.at[pl.ds(base, _BPW)], i_idx)
    pltpu.sync_copy(r_hbm.at[pl.ds(base, _BPW)], r_v)

    lanes = lax.iota(jnp.int32, 16)

    def row_grp(g, carry):
        jg = g * 16
        qu = u_idx[pl.ds(jg, 16)]
        qi = i_idx[pl.ds(jg, 16)]
        cps = []
        for l in range(16):
            bu = pl.multiple_of((qu[l] >> 3) * 8, 8)
            bi = pl.multiple_of((qi[l] >> 3) * 8, 8)
            cps.append(pltpu.async_copy(upk.at[pl.ds(bu, 8), :],
                                        u_rows.at[pl.ds(l * 8, 8), :], sem))
            cps.append(pltpu.async_copy(ipk.at[pl.ds(bi, 8), :],
                                        i_rows.at[pl.ds(l * 8, 8), :], sem))
        for cp in cps:
            cp.wait()
        for l in range(16):
            su = qu[l] & 7
            si = qi[l] & 7
            t = (u_rows[l * 8 + su, pl.ds(0, 16)]
                 * i_rows[l * 8 + si, pl.ds(0, 16)]
                 + u_rows[l * 8 + su, pl.ds(16, 16)]
                 * i_rows[l * 8 + si, pl.ds(16, 16)])
            plsc.store_scatter(tbuf, [lanes * _PITCH + jg + l], t)
        return carry

    lax.fori_loop(0, _BPW // 16, row_grp, jnp.int32(0))

    def grp_step(g, acc):
        g16 = g * 16
        rh = tbuf[pl.ds(g16, 16)]
        for l in range(1, 16):
            rh = rh + tbuf[pl.ds(l * _PITCH + g16, 16)]
        diff = rh - r_v[pl.ds(g16, 16)]
        return acc + diff * diff

    acc = lax.fori_loop(0, _BPW // 16, grp_step, jnp.zeros((16,), jnp.float32))

    stage_v[pl.ds(0, 16)] = acc
    for m in range(1, 8):
        stage_v[pl.ds(m * 16, 16)] = jnp.zeros((16,), jnp.float32)
    pltpu.sync_copy(stage_v, shared.at[s])
    plsc.subcore_barrier()

    @pl.when(s == 0)
    def _():
        pltpu.sync_copy(shared, parts_v)
        for seg in range(8):
            sl = pl.ds(seg * 16, 16)
            tot = parts_v[0, sl]
            for w in range(1, _NS):
                tot = tot + parts_v[w, sl]
            stage_v[sl] = tot
        pltpu.sync_copy(stage_v, out_hbm.at[c])


@functools.cache
def _mse_sc():
    return pl.kernel(
        _mse_sc_body,
        mesh=plsc.VectorSubcoreMesh(core_axis_name="c", subcore_axis_name="s"),
        out_type=jax.ShapeDtypeStruct((_NC, 128), jnp.float32),
        scratch_types=[
            pltpu.VMEM((_BPW,), jnp.int32),
            pltpu.VMEM((_BPW,), jnp.int32),
            pltpu.VMEM((_BPW,), jnp.float32),
            pltpu.VMEM((128, _D), jnp.float32),
            pltpu.VMEM((128, _D), jnp.float32),
            pltpu.VMEM((16 * _PITCH,), jnp.float32),
            pltpu.VMEM_SHARED((_NS, 128), jnp.float32),
            pltpu.VMEM((_NS, 128), jnp.float32),
            pltpu.VMEM((128,), jnp.float32),
            pltpu.SemaphoreType.DMA,
        ],
        compiler_params=pltpu.CompilerParams(
            needs_layout_passes=False, use_tc_tiling_on_sc=True),
    )


_BLKW = 8192


def _sq_tc_body(n_cols, x_ref, out_ref, acc_ref):
    k = pl.program_id(0)
    last = pl.num_programs(0) - 1

    @pl.when(k == 0)
    def _():
        acc_ref[...] = jnp.zeros((_D, 128), jnp.float32)

    def _block_sq(masked):
        accs = [None] * 4
        for j in range(_BLKW // 128):
            s = x_ref[:, pl.ds(j * 128, 128)]
            if masked:
                col = (lax.broadcasted_iota(jnp.int32, (_D, 128), 1)
                       + (k * _BLKW + j * 128))
                s = jnp.where(col < n_cols, s, jnp.float32(0.0))
            t = s * s
            m = j % 4
            accs[m] = t if accs[m] is None else accs[m] + t
        return (accs[0] + accs[1]) + (accs[2] + accs[3])

    @pl.when(k < last)
    def _():
        acc_ref[...] += _block_sq(False)

    @pl.when(k == last)
    def _():
        acc = acc_ref[...] + _block_sq(True)
        out_ref[...] = jnp.sum(acc).reshape(1, 1)

def _sq_tc(xt, n_cols):
    grid = (n_cols + _BLKW - 1) // _BLKW
    return pl.pallas_call(
        functools.partial(_sq_tc_body, n_cols),
        grid=(grid,),
        in_specs=[pl.BlockSpec((_D, _BLKW), lambda k: (0, k))],
        out_specs=pl.BlockSpec((1, 1), lambda k: (0, 0)),
        out_shape=jax.ShapeDtypeStruct((1, 1), jnp.float32),
        scratch_shapes=[pltpu.VMEM((_D, 128), jnp.float32)],
    )(xt)


def kernel(u, i, r, u_feat, i_feat):
    u = u.astype(jnp.int32)
    i = i.astype(jnp.int32)
    ufT = u_feat.T
    ifT = i_feat.T

    reg_u = _sq_tc(ufT, _NU)
    reg_i = _sq_tc(ifT, _NI)
    mse_parts = _mse_sc()(u, i, r, u_feat, i_feat)

    mse = jnp.sum(mse_parts)
    loss = mse + jnp.float32(_LAMBDA) * (reg_u[0, 0] + reg_i[0, 0])
    rmse = jnp.sqrt(mse * jnp.float32(1.0 / _BATCH))
    return (loss, rmse)


# speedup vs baseline: 1.3980x; 1.3980x over previous
import functools

import jax
import jax.numpy as jnp
from jax import lax
from jax.experimental import pallas as pl
from jax.experimental.pallas import tpu as pltpu
from jax.experimental.pallas import tpu_sc as plsc

_BATCH = 16384
_D = 32
_LAMBDA = 0.01
_NU = 1000000
_NI = 100000

_NC = 2
_NS = 16
_NW = _NC * _NS
_BPW = _BATCH // _NW
_GCHUNK = 128
_NCHUNK = _BPW // _GCHUNK
_PITCH = _BPW + 1


def _mse_sc_body(u_hbm, i_hbm, r_hbm, upk, ipk, out_hbm,
                 u_idx, i_idx, r_v, u_rows, i_rows, tbuf,
                 shared, parts_v, stage_v, sem):
    c = lax.axis_index("c")
    s = lax.axis_index("s")
    wid = c * _NS + s
    base = wid * _BPW

    pltpu.sync_copy(u_hbm.at[pl.ds(base, _BPW)], u_idx)
    pltpu.sync_copy(i_hbm.at[pl.ds(base, _BPW)], i_idx)
    pltpu.sync_copy(r_hbm.at[pl.ds(base, _BPW)], r_v)

    lanes = lax.iota(jnp.int32, 16)

    def pkrow(q):
        return ((q >> 13) << 11) | (q & 2047)

    for h in range(2):
        hb = h * 256
        cps = []
        for g in range(16):
            sl = pl.ds(hb + g * 16, 16)
            dsl = pl.ds(g * 16, 16)
            cps.append(pltpu.async_copy(upk.at[pkrow(u_idx[sl])],
                                        u_rows.at[dsl, :], sem))
            cps.append(pltpu.async_copy(ipk.at[pkrow(i_idx[sl])],
                                        i_rows.at[dsl, :], sem))
        for cp in cps:
            cp.wait()

        def row_grp(g, carry, hb=hb):
            jg = hb + g * 16
            lg = g * 16
            qu = u_idx[pl.ds(jg, 16)]
            qi = i_idx[pl.ds(jg, 16)]
            for l in range(16):
                jl = lg + l
                su = ((qu[l] >> 11) & 3) * 32
                si = ((qi[l] >> 11) & 3) * 32
                t = (u_rows[jl, pl.ds(su, 16)] * i_rows[jl, pl.ds(si, 16)]
                     + u_rows[jl, pl.ds(su + 16, 16)]
                     * i_rows[jl, pl.ds(si + 16, 16)])
                plsc.store_scatter(tbuf, [lanes * _PITCH + jg + l], t)
            return carry

        lax.fori_loop(0, 16, row_grp, jnp.int32(0))

    def grp_step(g, acc):
        g16 = g * 16
        rh = tbuf[pl.ds(g16, 16)]
        for l in range(1, 16):
            rh = rh + tbuf[pl.ds(l * _PITCH + g16, 16)]
        diff = rh - r_v[pl.ds(g16, 16)]
        return acc + diff * diff

    acc = lax.fori_loop(0, _BPW // 16, grp_step, jnp.zeros((16,), jnp.float32))

    stage_v[pl.ds(0, 16)] = acc
    for m in range(1, 8):
        stage_v[pl.ds(m * 16, 16)] = jnp.zeros((16,), jnp.float32)
    pltpu.sync_copy(stage_v, shared.at[s])
    plsc.subcore_barrier()

    @pl.when(s == 0)
    def _():
        pltpu.sync_copy(shared, parts_v)
        for seg in range(8):
            sl = pl.ds(seg * 16, 16)
            tot = parts_v[0, sl]
            for w in range(1, _NS):
                tot = tot + parts_v[w, sl]
            stage_v[sl] = tot
        pltpu.sync_copy(stage_v, out_hbm.at[c])


@functools.cache
def _mse_sc():
    return pl.kernel(
        _mse_sc_body,
        mesh=plsc.VectorSubcoreMesh(core_axis_name="c", subcore_axis_name="s"),
        out_type=jax.ShapeDtypeStruct((_NC, 128), jnp.float32),
        scratch_types=[
            pltpu.VMEM((_BPW,), jnp.int32),
            pltpu.VMEM((_BPW,), jnp.int32),
            pltpu.VMEM((_BPW,), jnp.float32),
            pltpu.VMEM((256, 4 * _D), jnp.float32),
            pltpu.VMEM((256, 4 * _D), jnp.float32),
            pltpu.VMEM((16 * _PITCH,), jnp.float32),
            pltpu.VMEM_SHARED((_NS, 128), jnp.float32),
            pltpu.VMEM((_NS, 128), jnp.float32),
            pltpu.VMEM((128,), jnp.float32),
            pltpu.SemaphoreType.DMA,
        ],
        compiler_params=pltpu.CompilerParams(
            needs_layout_passes=False, use_tc_tiling_on_sc=True),
    )


_BLKW = 8192


def _sq_tc_body(n_cols, x_ref, out_ref, pk_ref, acc_ref):
    k = pl.program_id(0)
    last = pl.num_programs(0) - 1

    @pl.when(k == 0)
    def _():
        acc_ref[...] = jnp.zeros((_D, 128), jnp.float32)

    def _block_sq(masked):
        accs = [None] * 4
        for j in range(_BLKW // 128):
            s = x_ref[:, pl.ds(j * 128, 128)]
            if masked:
                col = (lax.broadcasted_iota(jnp.int32, (_D, 128), 1)
                       + (k * _BLKW + j * 128))
                s = jnp.where(col < n_cols, s, jnp.float32(0.0))
            t = s * s
            m = j % 4
            accs[m] = t if accs[m] is None else accs[m] + t
        return (accs[0] + accs[1]) + (accs[2] + accs[3])

    @pl.when(k < last)
    def _():
        acc_ref[...] += _block_sq(False)

    @pl.when(k == last)
    def _():
        acc = acc_ref[...] + _block_sq(True)
        out_ref[...] = jnp.sum(acc).reshape(1, 1)

    x = x_ref[...]
    y = x.T
    qtr = _BLKW // 4
    pk_ref[...] = jnp.concatenate(
        [y[b * qtr:(b + 1) * qtr, :] for b in range(4)], axis=1)


def _sq_tc(xt, n_cols):
    grid = (n_cols + _BLKW - 1) // _BLKW
    return pl.pallas_call(
        functools.partial(_sq_tc_body, n_cols),
        grid=(grid,),
        in_specs=[pl.BlockSpec((_D, _BLKW), lambda k: (0, k))],
        out_specs=[
            pl.BlockSpec((1, 1), lambda k: (0, 0)),
            pl.BlockSpec((_BLKW // 4, 4 * _D), lambda k: (k, 0)),
        ],
        out_shape=[
            jax.ShapeDtypeStruct((1, 1), jnp.float32),
            jax.ShapeDtypeStruct((grid * _BLKW // 4, 4 * _D), jnp.float32),
        ],
        scratch_shapes=[pltpu.VMEM((_D, 128), jnp.float32)],
    )(xt)


def kernel(u, i, r, u_feat, i_feat):
    u = u.astype(jnp.int32)
    i = i.astype(jnp.int32)
    ufT = u_feat.T
    ifT = i_feat.T

    reg_u, upk = _sq_tc(ufT, _NU)
    reg_i, ipk = _sq_tc(ifT, _NI)
    mse_parts = _mse_sc()(u, i, r, upk, ipk)

    mse = jnp.sum(mse_parts)
    loss = mse + jnp.float32(_LAMBDA) * (reg_u[0, 0] + reg_i[0, 0])
    rmse = jnp.sqrt(mse * jnp.float32(1.0 / _BATCH))
    return (loss, rmse)


# speedup vs baseline: 1.6881x; 1.2075x over previous
import functools

import jax
import jax.numpy as jnp
from jax import lax
from jax.experimental import pallas as pl
from jax.experimental.pallas import tpu as pltpu
from jax.experimental.pallas import tpu_sc as plsc

_BATCH = 16384
_D = 32
_LAMBDA = 0.01
_NU = 1000000
_NI = 100000

_NC = 2
_NS = 16
_NW = _NC * _NS
_BPW = _BATCH // _NW
_GCHUNK = 128
_NCHUNK = _BPW // _GCHUNK
_PITCH = _BPW + 1


def _mse_sc_body(u_hbm, i_hbm, r_hbm, upk, ipk, out_hbm,
                 u_idx, i_idx, r_v, u_rows, i_rows, tbuf,
                 shared, parts_v, stage_v, sem):
    c = lax.axis_index("c")
    s = lax.axis_index("s")
    wid = c * _NS + s
    base = wid * _BPW

    pltpu.sync_copy(u_hbm.at[pl.ds(base, _BPW)], u_idx)
    pltpu.sync_copy(i_hbm.at[pl.ds(base, _BPW)], i_idx)
    pltpu.sync_copy(r_hbm.at[pl.ds(base, _BPW)], r_v)

    lanes = lax.iota(jnp.int32, 16)

    def pkrow(q):
        return ((q >> 13) << 11) | (q & 2047)

    for h in range(2):
        hb = h * 256
        cps = []
        for g in range(16):
            sl = pl.ds(hb + g * 16, 16)
            dsl = pl.ds(g * 16, 16)
            cps.append(pltpu.async_copy(upk.at[pkrow(u_idx[sl])],
                                        u_rows.at[dsl, :], sem))
            cps.append(pltpu.async_copy(ipk.at[pkrow(i_idx[sl])],
                                        i_rows.at[dsl, :], sem))
        for cp in cps:
            cp.wait()

        def row_grp(g, carry, hb=hb):
            jg = hb + g * 16
            lg = g * 16
            qu = u_idx[pl.ds(jg, 16)]
            qi = i_idx[pl.ds(jg, 16)]
            for l in range(16):
                jl = lg + l
                su = ((qu[l] >> 11) & 3) * 32
                si = ((qi[l] >> 11) & 3) * 32
                t = (u_rows[jl, pl.ds(su, 16)] * i_rows[jl, pl.ds(si, 16)]
                     + u_rows[jl, pl.ds(su + 16, 16)]
                     * i_rows[jl, pl.ds(si + 16, 16)])
                plsc.store_scatter(tbuf, [lanes * _PITCH + jg + l], t)
            return carry

        lax.fori_loop(0, 16, row_grp, jnp.int32(0))

    def grp_step(g, acc):
        g16 = g * 16
        rh = tbuf[pl.ds(g16, 16)]
        for l in range(1, 16):
            rh = rh + tbuf[pl.ds(l * _PITCH + g16, 16)]
        diff = rh - r_v[pl.ds(g16, 16)]
        return acc + diff * diff

    acc = lax.fori_loop(0, _BPW // 16, grp_step, jnp.zeros((16,), jnp.float32))

    stage_v[pl.ds(0, 16)] = acc
    for m in range(1, 8):
        stage_v[pl.ds(m * 16, 16)] = jnp.zeros((16,), jnp.float32)
    pltpu.sync_copy(stage_v, shared.at[s])
    plsc.subcore_barrier()

    @pl.when(s == 0)
    def _():
        pltpu.sync_copy(shared, parts_v)
        for seg in range(8):
            sl = pl.ds(seg * 16, 16)
            tot = parts_v[0, sl]
            for w in range(1, _NS):
                tot = tot + parts_v[w, sl]
            stage_v[sl] = tot
        pltpu.sync_copy(stage_v, out_hbm.at[c])


@functools.cache
def _mse_sc():
    return pl.kernel(
        _mse_sc_body,
        mesh=plsc.VectorSubcoreMesh(core_axis_name="c", subcore_axis_name="s"),
        out_type=jax.ShapeDtypeStruct((_NC, 128), jnp.float32),
        scratch_types=[
            pltpu.VMEM((_BPW,), jnp.int32),
            pltpu.VMEM((_BPW,), jnp.int32),
            pltpu.VMEM((_BPW,), jnp.float32),
            pltpu.VMEM((256, 4 * _D), jnp.float32),
            pltpu.VMEM((256, 4 * _D), jnp.float32),
            pltpu.VMEM((16 * _PITCH,), jnp.float32),
            pltpu.VMEM_SHARED((_NS, 128), jnp.float32),
            pltpu.VMEM((_NS, 128), jnp.float32),
            pltpu.VMEM((128,), jnp.float32),
            pltpu.SemaphoreType.DMA,
        ],
        compiler_params=pltpu.CompilerParams(
            needs_layout_passes=False, use_tc_tiling_on_sc=True),
    )


_BLKW = 8192


def _sq_tc_body(n_cols, x_ref, out_ref, pk_ref, acc_ref):
    k = pl.program_id(0)
    last = pl.num_programs(0) - 1

    @pl.when(k == 0)
    def _():
        acc_ref[...] = jnp.zeros((_D, 128), jnp.float32)

    def _block_sq(masked):
        accs = [None] * 4
        for j in range(_BLKW // 128):
            s = x_ref[:, pl.ds(j * 128, 128)]
            if masked:
                col = (lax.broadcasted_iota(jnp.int32, (_D, 128), 1)
                       + (k * _BLKW + j * 128))
                s = jnp.where(col < n_cols, s, jnp.float32(0.0))
            t = s * s
            m = j % 4
            accs[m] = t if accs[m] is None else accs[m] + t
        return (accs[0] + accs[1]) + (accs[2] + accs[3])

    @pl.when(k < last)
    def _():
        acc_ref[...] += _block_sq(False)

    @pl.when(k == last)
    def _():
        acc = acc_ref[...] + _block_sq(True)
        out_ref[...] = jnp.sum(acc).reshape(1, 1)

    x = x_ref[...]
    eye = jnp.eye(_D, dtype=jnp.bfloat16)
    y = lax.dot_general(x.astype(jnp.bfloat16), eye,
                        (((0,), (0,)), ((), ())),
                        preferred_element_type=jnp.float32)
    qtr = _BLKW // 4
    pk_ref[...] = jnp.concatenate(
        [y[b * qtr:(b + 1) * qtr, :] for b in range(4)], axis=1)


def _sq_tc(xt, n_cols):
    grid = (n_cols + _BLKW - 1) // _BLKW
    return pl.pallas_call(
        functools.partial(_sq_tc_body, n_cols),
        grid=(grid,),
        in_specs=[pl.BlockSpec((_D, _BLKW), lambda k: (0, k))],
        out_specs=[
            pl.BlockSpec((1, 1), lambda k: (0, 0)),
            pl.BlockSpec((_BLKW // 4, 4 * _D), lambda k: (k, 0)),
        ],
        out_shape=[
            jax.ShapeDtypeStruct((1, 1), jnp.float32),
            jax.ShapeDtypeStruct((grid * _BLKW // 4, 4 * _D), jnp.float32),
        ],
        scratch_shapes=[pltpu.VMEM((_D, 128), jnp.float32)],
    )(xt)


def kernel(u, i, r, u_feat, i_feat):
    u = u.astype(jnp.int32)
    i = i.astype(jnp.int32)
    ufT = u_feat.T
    ifT = i_feat.T

    reg_u, upk = _sq_tc(ufT, _NU)
    reg_i, ipk = _sq_tc(ifT, _NI)
    mse_parts = _mse_sc()(u, i, r, upk, ipk)

    mse = jnp.sum(mse_parts)
    loss = mse + jnp.float32(_LAMBDA) * (reg_u[0, 0] + reg_i[0, 0])
    rmse = jnp.sqrt(mse * jnp.float32(1.0 / _BATCH))
    return (loss, rmse)
